# baseline (speedup 1.0000x reference)
import functools

import jax
import jax.numpy as jnp
from jax import lax
from jax.experimental import pallas as pl
from jax.experimental.pallas import tpu as pltpu
from jax.experimental.pallas import tpu_sc as plsc

PR, PC = 5, 5
NR, NC = 128, 128
NB = 40
ALPHA = 15.0
Q = PR * PC
W1C = NC * PC
NBP = 48

_KS = 16

L = 16
NWORK = 32
CPW = NC // 2
RPW = PR
WCW = CPW * PC
HC = CPW // 2

TN = NR - _KS
TROWS = TN * PR


def _splat(v):
    return jnp.full((L,), v, jnp.int32)


def _sc_body(wd_h, big_h, bnd_h, out_h,
             w1b, u1f, dvb, nsb, npb, vxb, vyb, uxb, uyb,
             s1b, accb, cb, bndb, cidx, sems, osem):
    wid = lax.axis_index("s") * 2 + lax.axis_index("c")
    iota = lax.iota(jnp.int32, L)

    pltpu.sync_copy(bnd_h, bndb)

    ii = wid // 2
    jh = wid % 2
    r0 = ii * PR
    c0 = ii * NC + jh * CPW
    w0 = jh * WCW

    cp_w1 = pltpu.async_copy(wd_h.at[pl.ds(r0, RPW), pl.ds(w0, WCW)],
                             w1b, sems.at[0])
    cp_ns = pltpu.async_copy(big_h.at[pl.ds(0, Q), pl.ds(c0, HC)],
                             nsb, sems.at[1])
    cp_np = pltpu.async_copy(big_h.at[pl.ds(Q, Q), pl.ds(c0, HC)],
                             npb, sems.at[2])
    cp_vx = pltpu.async_copy(big_h.at[pl.ds(2 * Q, NB), pl.ds(c0, CPW)],
                             vxb, sems.at[3])
    cp_vy = pltpu.async_copy(big_h.at[pl.ds(2 * Q + NB, NB), pl.ds(c0, CPW)],
                             vyb, sems.at[4])
    cp_s1 = pltpu.async_copy(big_h.at[pl.ds(2 * Q + 2 * NB, 1),
                                      pl.ds(c0, CPW)], s1b, sems.at[5])
    cp_dv = pltpu.async_copy(wd_h.at[pl.ds(_KS * PR + r0, RPW),
                                     pl.ds(w0, WCW)], dvb, sems.at[6])

    @plsc.parallel_loop(0, WCW // L, 1, unroll=2)
    def cidx_ch(ch):
        cidx[pl.ds(ch * L, L)] = (ch * L + iota) // PC

    cp_w1.wait()
    for h in range(2):
        if h == 0:
            cp_ns.wait()
            cp_np.wait()
        else:
            cp_ns2.wait()
            cp_np2.wait()

        @plsc.parallel_loop(0, HC // L, 1, unroll=2)
        def build_ch(ch):
            gch = h * (HC // L) + ch
            jjv = gch * L + iota
            a = jnp.zeros((L,), jnp.float32)
            for q in range(Q):
                rowv = _splat(q % PR)
                colv = jjv * PC + (q // PR)
                val = plsc.load_gather(w1b, [rowv, colv])
                u1f[pl.ds(q * CPW + gch * L, L)] = val
                a = a + (nsb[q, pl.ds(ch * L, L)] - val) * npb[q, pl.ds(ch * L, L)]
            accb[pl.ds(gch * L, L)] = a
            cb[pl.ds(gch * L, L)] = jnp.zeros((L,), jnp.float32)

        if h == 0:
            cp_ns2 = pltpu.async_copy(big_h.at[pl.ds(0, Q),
                                               pl.ds(c0 + HC, HC)],
                                      nsb, sems.at[1])
            cp_np2 = pltpu.async_copy(big_h.at[pl.ds(Q, Q),
                                               pl.ds(c0 + HC, HC)],
                                      npb, sems.at[2])

    def bld_row(b, _):
        ox0 = plsc.load_gather(bndb, [_splat(0), _splat(b)]) * CPW
        ox1 = plsc.load_gather(bndb, [_splat(1), _splat(b)]) * CPW
        oy0 = plsc.load_gather(bndb, [_splat(2), _splat(b)]) * CPW
        oy1 = plsc.load_gather(bndb, [_splat(3), _splat(b)]) * CPW

        @plsc.parallel_loop(0, CPW // L, 1, unroll=4)
        def bld_ch(ch):
            colv = ch * L + iota
            sl = pl.ds(ch * L, L)
            uxb[b, sl] = (plsc.load_gather(u1f, [ox0 + colv])
                          - plsc.load_gather(u1f, [ox1 + colv]))
            uyb[b, sl] = (plsc.load_gather(u1f, [oy0 + colv])
                          - plsc.load_gather(u1f, [oy1 + colv]))
        return 0
    lax.fori_loop(0, NB, bld_row, 0, unroll=False)

    cp_vx.wait()
    cp_vy.wait()
    cp_s1.wait()

    def fp(it, _):
        @plsc.parallel_loop(0, CPW // L, 1, unroll=2)
        def ch_body(ch):
            sl = pl.ds(ch * L, L)
            cv = cb[sl]
            def row(b, carry):
                f, s = carry
                uxv = uxb[b, sl]
                uyv = uyb[b, sl]
                vxv = vxb[b, sl]
                vyv = vyb[b, sl]
                ucx = uxv + cv * vxv
                ucy = uyv + cv * vyv
                u2 = ucx * ucx + ucy * ucy + 1e-4
                ib = plsc.bitcast(u2, jnp.int32)
                y = plsc.bitcast(jnp.int32(0x5F3759DF) - (ib >> 1),
                                 jnp.float32)
                y = y * (1.5 - (0.5 * u2) * y * y)
                f = f + (vxv * vxv + vyv * vyv) * y
                s = s + (uxv * vxv + uyv * vyv) * y
                return f, s
            z = jnp.zeros((L,), jnp.float32)
            f, s = lax.fori_loop(0, NB, row, (z, z), unroll=False)
            cb[sl] = (accb[sl] - ALPHA * s) / (ALPHA * f + s1b[0, sl])
        return 0
    lax.fori_loop(0, 6, fp, 0, unroll=False)

    cp_dv.wait()
    for rl in range(RPW):

        @plsc.parallel_loop(0, WCW // L, 1, unroll=4)
        def out_ch(ch):
            sl = pl.ds(ch * L, L)
            cg = plsc.load_gather(cb, [cidx[sl]])
            w1b[rl, sl] = w1b[rl, sl] + cg * dvb[rl, sl]

    pltpu.async_copy(w1b, out_h.at[pl.ds(r0, RPW), pl.ds(w0, WCW)],
                     osem).wait()


def _make_sc():
    mesh = plsc.VectorSubcoreMesh(core_axis_name="c", subcore_axis_name="s",
                                  num_cores=2, num_subcores=16)
    return pl.kernel(
        _sc_body,
        out_type=jax.ShapeDtypeStruct((_KS * PR, W1C), jnp.float32),
        mesh=mesh,
        compiler_params=pltpu.CompilerParams(use_tc_tiling_on_sc=False,
                                             needs_layout_passes=False),
        scratch_types=[
            pltpu.VMEM((RPW, WCW), jnp.float32),
            pltpu.VMEM((Q * CPW,), jnp.float32),
            pltpu.VMEM((RPW, WCW), jnp.float32),
            pltpu.VMEM((Q, HC), jnp.float32),
            pltpu.VMEM((Q, HC), jnp.float32),
            pltpu.VMEM((NB, CPW), jnp.float32),
            pltpu.VMEM((NB, CPW), jnp.float32),
            pltpu.VMEM((NB, CPW), jnp.float32),
            pltpu.VMEM((NB, CPW), jnp.float32),
            pltpu.VMEM((1, CPW), jnp.float32),
            pltpu.VMEM((CPW,), jnp.float32),
            pltpu.VMEM((CPW,), jnp.float32),
            pltpu.VMEM((4, NBP), jnp.int32),
            pltpu.VMEM((WCW,), jnp.int32),
            pltpu.SemaphoreType.DMA((7,)),
            pltpu.SemaphoreType.DMA,
        ],
    )


BI = 16
NSTEP = TN // BI
OFF = _KS // BI
BC = BI * NC
BR = BI * PR


def _tc_body(w1r, nsr, npr, vxr, vyr, dvr, s1r, sxr, syr, cselr, browr,
             bcolr, outr, uscr, cscr):
    w1v = w1r[...]
    for g in range(PC):
        csel = cselr[:, g * NC:(g + 1) * NC]
        ag = jnp.dot(w1v, csel, preferred_element_type=jnp.float32)
        for il in range(BI):
            uscr[pl.ds(g * PR, PR), pl.ds(il * NC, NC)] = (
                ag[il * PR:(il + 1) * PR, :])

    uall = uscr[...]
    acc = jnp.sum((nsr[...] - uall) * npr[...], axis=0, keepdims=True)

    ux = jnp.dot(sxr[...], uall, preferred_element_type=jnp.float32)
    uy = jnp.dot(syr[...], uall, preferred_element_type=jnp.float32)

    vxv = vxr[...]
    vyv = vyr[...]
    a1 = vxv * vxv + vyv * vyv
    c1 = ux * vxv + uy * vyv
    s1v = s1r[...].reshape(1, BC)

    c = jnp.zeros((1, BC), jnp.float32)
    for _ in range(6):
        ucx = ux + c * vxv
        ucy = uy + c * vyv
        rb = lax.rsqrt(ucx * ucx + ucy * ucy + 1e-4)
        firs = jnp.sum(a1 * rb, axis=0, keepdims=True)
        sec = jnp.sum(c1 * rb, axis=0, keepdims=True)
        c = (acc - ALPHA * sec) / (ALPHA * firs + s1v)

    for il in range(BI):
        cscr[il, :] = c[0, il * NC:(il + 1) * NC]
    dc = jnp.dot(jnp.dot(browr[...], cscr[...],
                         preferred_element_type=jnp.float32),
                 bcolr[...], preferred_element_type=jnp.float32)
    outr[...] = w1v + dc * dvr[...]


def _make_tc():
    return pl.pallas_call(
        _tc_body,
        grid=(NSTEP,),
        in_specs=[
            pl.BlockSpec((BR, W1C), lambda i: (OFF + i, 0)),
            pl.BlockSpec((Q, BC), lambda i: (0, OFF + i)),
            pl.BlockSpec((Q, BC), lambda i: (0, OFF + i)),
            pl.BlockSpec((NB, BC), lambda i: (0, OFF + i)),
            pl.BlockSpec((NB, BC), lambda i: (0, OFF + i)),
            pl.BlockSpec((BR, W1C), lambda i: (OFF + i, 0)),
            pl.BlockSpec((BC,), lambda i: (OFF + i,)),
            pl.BlockSpec((NB, Q), lambda i: (0, 0)),
            pl.BlockSpec((NB, Q), lambda i: (0, 0)),
            pl.BlockSpec((W1C, PC * NC), lambda i: (0, 0)),
            pl.BlockSpec((BR, BI), lambda i: (0, 0)),
            pl.BlockSpec((NC, W1C), lambda i: (0, 0)),
        ],
        out_specs=pl.BlockSpec((BR, W1C), lambda i: (i, 0)),
        out_shape=jax.ShapeDtypeStruct((TN * PR, W1C), jnp.float32),
        scratch_shapes=[pltpu.VMEM((Q, BC), jnp.float32),
                        pltpu.VMEM((BI, NC), jnp.float32)],
        compiler_params=pltpu.CompilerParams(
            dimension_semantics=("arbitrary",),
            vmem_limit_bytes=110 * 1024 * 1024),
    )


def kernel(w1, noise_ch, nphi, vx, vy, dv1, s1, bnd_idx, bnd_idy):
    bnd = jnp.pad(jnp.concatenate([bnd_idx.astype(jnp.int32).T,
                                   bnd_idy.astype(jnp.int32).T], axis=0),
                  ((0, 0), (0, NBP - NB)))

    r0 = _KS * PR
    c0 = _KS * NC
    wd = jnp.concatenate([w1[:r0], dv1[:r0]], axis=0)
    big = jnp.concatenate([noise_ch[:, :c0], nphi[:, :c0], vx[:, :c0],
                           vy[:, :c0], s1[None, :c0]], axis=0)
    sc_out = _make_sc()(wd, big, bnd)
    qio = jnp.arange(Q, dtype=jnp.int32)[None, :]
    sx = ((qio == bnd[0, :NB, None]).astype(jnp.float32)
          - (qio == bnd[1, :NB, None]).astype(jnp.float32))
    sy = ((qio == bnd[2, :NB, None]).astype(jnp.float32)
          - (qio == bnd[3, :NB, None]).astype(jnp.float32))
    ci = jnp.arange(W1C, dtype=jnp.int32)
    jg = jnp.arange(PC * NC, dtype=jnp.int32)
    cselall = (ci[:, None] == (jg % NC) * PC + jg // NC).astype(jnp.float32)
    brow = (jnp.arange(BR, dtype=jnp.int32)[:, None] // PR
            == jnp.arange(BI, dtype=jnp.int32)[None, :]).astype(jnp.float32)
    bcol = (jnp.arange(W1C, dtype=jnp.int32)[None, :] // PC
            == jnp.arange(NC, dtype=jnp.int32)[:, None]).astype(jnp.float32)
    tc_out = _make_tc()(w1, noise_ch, nphi, vx, vy, dv1, s1, sx, sy,
                        cselall, brow, bcol)
    return jnp.concatenate([sc_out, tc_out], axis=0)


# speedup vs baseline: 1.0883x; 1.0883x over previous
import jax
import jax.numpy as jnp
from jax import lax
from jax.experimental import pallas as pl
from jax.experimental.pallas import tpu as pltpu
from jax.experimental.pallas import tpu_sc as plsc

PR, PC = 5, 5
NR, NC = 128, 128
NB = 40
ALPHA = 15.0
Q = PR * PC
W1C = NC * PC
NBP = 48

_KS = 16

L = 16
NWORK = 32
CPW = NC // 2
RPW = PR
WCW = CPW * PC
HC = CPW // 2

TN = NR - _KS
TROWS = TN * PR


def _splat(v):
    return jnp.full((L,), v, jnp.int32)


def _sc_body(wd_h, big_h, bnd_h, out_h,
             w1b, u1f, dvb, nsb, npb, vxb, vyb, uxb, uyb,
             s1b, accb, cb, bndb, cidx, sems, osem):
    wid = lax.axis_index("s") * 2 + lax.axis_index("c")
    iota = lax.iota(jnp.int32, L)

    pltpu.sync_copy(bnd_h, bndb)

    ii = wid // 2
    jh = wid % 2
    r0 = ii * PR
    c0 = ii * NC + jh * CPW
    w0 = jh * WCW

    cp_w1 = pltpu.async_copy(wd_h.at[pl.ds(r0, RPW), pl.ds(w0, WCW)],
                             w1b, sems.at[0])
    cp_ns = pltpu.async_copy(big_h.at[pl.ds(0, Q), pl.ds(c0, HC)],
                             nsb, sems.at[1])
    cp_np = pltpu.async_copy(big_h.at[pl.ds(Q, Q), pl.ds(c0, HC)],
                             npb, sems.at[2])
    cp_vx = pltpu.async_copy(big_h.at[pl.ds(2 * Q, NB), pl.ds(c0, CPW)],
                             vxb, sems.at[3])
    cp_vy = pltpu.async_copy(big_h.at[pl.ds(2 * Q + NB, NB), pl.ds(c0, CPW)],
                             vyb, sems.at[4])
    cp_s1 = pltpu.async_copy(big_h.at[pl.ds(2 * Q + 2 * NB, 1),
                                      pl.ds(c0, CPW)], s1b, sems.at[5])
    cp_dv = pltpu.async_copy(wd_h.at[pl.ds(_KS * PR + r0, RPW),
                                     pl.ds(w0, WCW)], dvb, sems.at[6])

    @plsc.parallel_loop(0, WCW // L, 1, unroll=2)
    def cidx_ch(ch):
        cidx[pl.ds(ch * L, L)] = (ch * L + iota) // PC

    cp_w1.wait()
    for h in range(2):
        if h == 0:
            cp_ns.wait()
            cp_np.wait()
        else:
            cp_ns2.wait()
            cp_np2.wait()

        @plsc.parallel_loop(0, HC // L, 1, unroll=2)
        def build_ch(ch):
            gch = h * (HC // L) + ch
            jjv = gch * L + iota
            a = jnp.zeros((L,), jnp.float32)
            for q in range(Q):
                rowv = _splat(q % PR)
                colv = jjv * PC + (q // PR)
                val = plsc.load_gather(w1b, [rowv, colv])
                u1f[pl.ds(q * CPW + gch * L, L)] = val
                a = a + (nsb[q, pl.ds(ch * L, L)] - val) * npb[q, pl.ds(ch * L, L)]
            accb[pl.ds(gch * L, L)] = a
            cb[pl.ds(gch * L, L)] = jnp.zeros((L,), jnp.float32)

        if h == 0:
            cp_ns2 = pltpu.async_copy(big_h.at[pl.ds(0, Q),
                                               pl.ds(c0 + HC, HC)],
                                      nsb, sems.at[1])
            cp_np2 = pltpu.async_copy(big_h.at[pl.ds(Q, Q),
                                               pl.ds(c0 + HC, HC)],
                                      npb, sems.at[2])

    def bld_row(b, _):
        ox0 = plsc.load_gather(bndb, [_splat(0), _splat(b)]) * CPW
        ox1 = plsc.load_gather(bndb, [_splat(1), _splat(b)]) * CPW
        oy0 = plsc.load_gather(bndb, [_splat(2), _splat(b)]) * CPW
        oy1 = plsc.load_gather(bndb, [_splat(3), _splat(b)]) * CPW

        @plsc.parallel_loop(0, CPW // L, 1, unroll=4)
        def bld_ch(ch):
            colv = ch * L + iota
            sl = pl.ds(ch * L, L)
            uxb[b, sl] = (plsc.load_gather(u1f, [ox0 + colv])
                          - plsc.load_gather(u1f, [ox1 + colv]))
            uyb[b, sl] = (plsc.load_gather(u1f, [oy0 + colv])
                          - plsc.load_gather(u1f, [oy1 + colv]))
        return 0
    lax.fori_loop(0, NB, bld_row, 0, unroll=False)

    cp_vx.wait()
    cp_vy.wait()
    cp_s1.wait()

    def fp(it, _):
        @plsc.parallel_loop(0, CPW // L, 1, unroll=2)
        def ch_body(ch):
            sl = pl.ds(ch * L, L)
            cv = cb[sl]
            def row(b, carry):
                f, s = carry
                uxv = uxb[b, sl]
                uyv = uyb[b, sl]
                vxv = vxb[b, sl]
                vyv = vyb[b, sl]
                ucx = uxv + cv * vxv
                ucy = uyv + cv * vyv
                u2 = ucx * ucx + ucy * ucy + 1e-4
                ib = plsc.bitcast(u2, jnp.int32)
                y = plsc.bitcast(jnp.int32(0x5F3759DF) - (ib >> 1),
                                 jnp.float32)
                y = y * (1.5 - (0.5 * u2) * y * y)
                f = f + (vxv * vxv + vyv * vyv) * y
                s = s + (uxv * vxv + uyv * vyv) * y
                return f, s
            z = jnp.zeros((L,), jnp.float32)
            f, s = lax.fori_loop(0, NB, row, (z, z), unroll=False)
            cb[sl] = (accb[sl] - ALPHA * s) / (ALPHA * f + s1b[0, sl])
        return 0
    lax.fori_loop(0, 6, fp, 0, unroll=False)

    cp_dv.wait()
    for rl in range(RPW):

        @plsc.parallel_loop(0, WCW // L, 1, unroll=4)
        def out_ch(ch):
            sl = pl.ds(ch * L, L)
            cg = plsc.load_gather(cb, [cidx[sl]])
            w1b[rl, sl] = w1b[rl, sl] + cg * dvb[rl, sl]

    pltpu.async_copy(w1b, out_h.at[pl.ds(r0, RPW), pl.ds(w0, WCW)],
                     osem).wait()


def _make_sc():
    mesh = plsc.VectorSubcoreMesh(core_axis_name="c", subcore_axis_name="s",
                                  num_cores=2, num_subcores=16)
    return pl.kernel(
        _sc_body,
        out_type=jax.ShapeDtypeStruct((_KS * PR, W1C), jnp.float32),
        mesh=mesh,
        compiler_params=pltpu.CompilerParams(use_tc_tiling_on_sc=False,
                                             needs_layout_passes=False),
        scratch_types=[
            pltpu.VMEM((RPW, WCW), jnp.float32),
            pltpu.VMEM((Q * CPW,), jnp.float32),
            pltpu.VMEM((RPW, WCW), jnp.float32),
            pltpu.VMEM((Q, HC), jnp.float32),
            pltpu.VMEM((Q, HC), jnp.float32),
            pltpu.VMEM((NB, CPW), jnp.float32),
            pltpu.VMEM((NB, CPW), jnp.float32),
            pltpu.VMEM((NB, CPW), jnp.float32),
            pltpu.VMEM((NB, CPW), jnp.float32),
            pltpu.VMEM((1, CPW), jnp.float32),
            pltpu.VMEM((CPW,), jnp.float32),
            pltpu.VMEM((CPW,), jnp.float32),
            pltpu.VMEM((4, NBP), jnp.int32),
            pltpu.VMEM((WCW,), jnp.int32),
            pltpu.SemaphoreType.DMA((7,)),
            pltpu.SemaphoreType.DMA,
        ],
    )


BI = 16
NSTEP = TN // BI
OFF = _KS // BI
BC = BI * NC
BR = BI * PR


def _tc_body(w1r, nsr, npr, vxr, vyr, dvr, s1r, sxr, syr, outr, uscr, cscr):
    cidx = lax.broadcasted_iota(jnp.int32, (W1C, NC), 0)
    jidx = lax.broadcasted_iota(jnp.int32, (W1C, NC), 1)
    w1v = w1r[...]
    for g in range(PC):
        csel = (cidx == jidx * PC + g).astype(jnp.float32)
        ag = jnp.dot(w1v, csel, preferred_element_type=jnp.float32)
        for il in range(BI):
            uscr[pl.ds(g * PR, PR), pl.ds(il * NC, NC)] = (
                ag[il * PR:(il + 1) * PR, :])

    uall = uscr[...]
    acc = jnp.sum((nsr[...] - uall) * npr[...], axis=0, keepdims=True)

    ux = jnp.dot(sxr[...], uall, preferred_element_type=jnp.float32)
    uy = jnp.dot(syr[...], uall, preferred_element_type=jnp.float32)

    vxv = vxr[...]
    vyv = vyr[...]
    a1 = vxv * vxv + vyv * vyv
    c1 = ux * vxv + uy * vyv
    s1v = s1r[...].reshape(1, BC)

    c = jnp.zeros((1, BC), jnp.float32)
    for _ in range(6):
        ucx = ux + c * vxv
        ucy = uy + c * vyv
        rb = lax.rsqrt(ucx * ucx + ucy * ucy + 1e-4)
        firs = jnp.sum(a1 * rb, axis=0, keepdims=True)
        sec = jnp.sum(c1 * rb, axis=0, keepdims=True)
        c = (acc - ALPHA * sec) / (ALPHA * firs + s1v)

    for il in range(BI):
        cscr[il, :] = c[0, il * NC:(il + 1) * NC]
    brow = (lax.broadcasted_iota(jnp.int32, (BR, BI), 0) // PR
            == lax.broadcasted_iota(jnp.int32, (BR, BI), 1)
            ).astype(jnp.float32)
    bcol = (lax.broadcasted_iota(jnp.int32, (NC, W1C), 1) // PC
            == lax.broadcasted_iota(jnp.int32, (NC, W1C), 0)
            ).astype(jnp.float32)
    dc = jnp.dot(jnp.dot(brow, cscr[...], preferred_element_type=jnp.float32),
                 bcol, preferred_element_type=jnp.float32)
    outr[...] = w1v + dc * dvr[...]


def _make_tc():
    return pl.pallas_call(
        _tc_body,
        grid=(NSTEP,),
        in_specs=[
            pl.BlockSpec((BR, W1C), lambda i: (OFF + i, 0)),
            pl.BlockSpec((Q, BC), lambda i: (0, OFF + i)),
            pl.BlockSpec((Q, BC), lambda i: (0, OFF + i)),
            pl.BlockSpec((NB, BC), lambda i: (0, OFF + i)),
            pl.BlockSpec((NB, BC), lambda i: (0, OFF + i)),
            pl.BlockSpec((BR, W1C), lambda i: (OFF + i, 0)),
            pl.BlockSpec((BC,), lambda i: (OFF + i,)),
            pl.BlockSpec((NB, Q), lambda i: (0, 0)),
            pl.BlockSpec((NB, Q), lambda i: (0, 0)),
        ],
        out_specs=pl.BlockSpec((BR, W1C), lambda i: (i, 0)),
        out_shape=jax.ShapeDtypeStruct((TN * PR, W1C), jnp.float32),
        scratch_shapes=[pltpu.VMEM((Q, BC), jnp.float32),
                        pltpu.VMEM((BI, NC), jnp.float32)],
        compiler_params=pltpu.CompilerParams(
            dimension_semantics=("arbitrary",),
            vmem_limit_bytes=110 * 1024 * 1024),
    )


def kernel(w1, noise_ch, nphi, vx, vy, dv1, s1, bnd_idx, bnd_idy):
    bnd = jnp.pad(jnp.concatenate([bnd_idx.astype(jnp.int32).T,
                                   bnd_idy.astype(jnp.int32).T], axis=0),
                  ((0, 0), (0, NBP - NB)))

    r0 = _KS * PR
    c0 = _KS * NC
    wd = jnp.concatenate([w1[:r0], dv1[:r0]], axis=0)
    big = jnp.concatenate([noise_ch[:, :c0], nphi[:, :c0], vx[:, :c0],
                           vy[:, :c0], s1[None, :c0]], axis=0)
    sc_out = _make_sc()(wd, big, bnd)
    qio = jnp.arange(Q, dtype=jnp.int32)[None, :]
    sx = ((qio == bnd[0, :NB, None]).astype(jnp.float32)
          - (qio == bnd[1, :NB, None]).astype(jnp.float32))
    sy = ((qio == bnd[2, :NB, None]).astype(jnp.float32)
          - (qio == bnd[3, :NB, None]).astype(jnp.float32))
    tc_out = _make_tc()(w1, noise_ch, nphi, vx, vy, dv1, s1, sx, sy)
    return jnp.concatenate([sc_out, tc_out], axis=0)
